# baseline (speedup 1.0000x reference)
import functools

import jax
import jax.numpy as jnp
from jax import lax
from jax.experimental import pallas as pl
from jax.experimental.pallas import tpu as pltpu
from jax.experimental.pallas import tpu_sc as plsc

BATCH = 4096
FIELDS = 26
EMB_DIM = 32
NUM_IDX = BATCH * FIELDS

NC = 2
NS = 16
NW = NC * NS
B_PER_W = NUM_IDX // NW


def _make_gather():
  mesh = plsc.VectorSubcoreMesh(core_axis_name="c", subcore_axis_name="s")

  @functools.partial(
      pl.kernel,
      mesh=mesh,
      out_type=jax.ShapeDtypeStruct((NUM_IDX, EMB_DIM), jnp.float32),
      compiler_params=pltpu.CompilerParams(use_tc_tiling_on_sc=False),
      scratch_types=[
          pltpu.VMEM((B_PER_W,), jnp.int32),
          pltpu.VMEM((B_PER_W, EMB_DIM), jnp.float32),
          pltpu.SemaphoreType.DMA,
      ],
  )
  def gather(idx_hbm, table_hbm, out_hbm, idx_v, rows_v, sem):
    wid = lax.axis_index("s") * NC + lax.axis_index("c")
    base = wid * B_PER_W
    pltpu.sync_copy(idx_hbm.at[pl.ds(base, B_PER_W)], idx_v)
    pltpu.async_copy(table_hbm.at[idx_v], rows_v, sem).wait()
    pltpu.sync_copy(rows_v, out_hbm.at[pl.ds(base, B_PER_W)])

  return gather


_gather = _make_gather()


@jax.jit
def kernel(x, weight):
  idx = x.reshape(NUM_IDX).astype(jnp.int32)
  out = _gather(idx, weight)
  return out.reshape(BATCH, FIELDS, EMB_DIM)
